# baseline (speedup 1.0000x reference)
import functools

import jax
import jax.numpy as jnp
from jax import lax
from jax.experimental import pallas as pl
from jax.experimental.pallas import tpu as pltpu
from jax.experimental.pallas import tpu_sc as plsc

N = 10000
D = 128
NP = 10240
NC, NS = 2, 16
NW = NC * NS
CH = 96
RT = NP // NS
ZB = 64
BM = 512


def _f32(*shape):
    return jax.ShapeDtypeStruct(shape, jnp.float32)


def _edge_kernel(n_chunks, width, heads):
    mesh = plsc.VectorSubcoreMesh(core_axis_name="c", subcore_axis_name="s")

    @functools.partial(
        pl.kernel,
        out_type=[_f32(NC, NP, width), _f32(NC, NP, 16)],
        mesh=mesh,
        compiler_params=pltpu.CompilerParams(
            needs_layout_passes=False, use_tc_tiling_on_sc=False),
        scratch_types=[
            pltpu.VMEM_SHARED((NP, width), jnp.float32),
            pltpu.VMEM_SHARED((NP, 16), jnp.float32),
            pltpu.VMEM((3, 2, CH), jnp.int32),
            pltpu.VMEM((2, CH, width), jnp.float32),
            pltpu.VMEM((2, CH, 16), jnp.float32),
            pltpu.VMEM((2, CH, 16), jnp.float32),
            pltpu.VMEM((2, CH, 16), jnp.float32),
            pltpu.SemaphoreType.DMA,
            pltpu.SemaphoreType.DMA,
            pltpu.SemaphoreType.DMA,
        ],
    )
    def kfn(xw_hbm, att_hbm, srcs_hbm, dsts_hbm, num_out, den_out,
            acc_num, acc_den, idx_v, rows_v, atts_v, attd_v, ex_v,
            sem_g, sem_s, sem_i):
        c = lax.axis_index("c")
        s = lax.axis_index("s")
        w = s * NC + c
        K = n_chunks
        zero16 = jnp.zeros((16,), jnp.float32)
        lanes = lax.iota(jnp.int32, 16)

        def _zrow(r, _):
            for j in range(width // 16):
                rows_v[0, r, pl.ds(j * 16, 16)] = zero16
            ex_v[0, r, pl.ds(0, 16)] = zero16
            ex_v[1, r, pl.ds(0, 16)] = zero16
            return 0
        lax.fori_loop(0, CH, _zrow, 0)

        for i in range(RT // ZB):
            base = s * RT + i * ZB
            pltpu.sync_copy(rows_v.at[0, pl.ds(0, ZB)],
                            acc_num.at[pl.ds(base, ZB)])
            pltpu.sync_copy(ex_v.at[0, pl.ds(0, ZB)],
                            acc_den.at[pl.ds(base, ZB)])
        plsc.subcore_barrier()

        def issue_idx(jj, slot):
            pltpu.async_copy(srcs_hbm.at[w, jj], idx_v.at[slot, 0], sem_i)
            pltpu.async_copy(dsts_hbm.at[w, jj], idx_v.at[slot, 1], sem_i)

        def drain_idx(slot):
            pltpu.make_async_copy(srcs_hbm.at[w, 0], idx_v.at[slot, 0],
                                  sem_i).wait()
            pltpu.make_async_copy(srcs_hbm.at[w, 0], idx_v.at[slot, 1],
                                  sem_i).wait()

        H1 = CH // 2

        def issue_gathers(slot, p):
            pltpu.async_copy(xw_hbm.at[idx_v.at[slot, 0, pl.ds(0, H1)]],
                             rows_v.at[p, pl.ds(0, H1)], sem_g)
            pltpu.async_copy(xw_hbm.at[idx_v.at[slot, 0, pl.ds(H1, H1)]],
                             rows_v.at[p, pl.ds(H1, H1)], sem_g)
            pltpu.async_copy(att_hbm.at[idx_v.at[slot, 0]], atts_v.at[p],
                             sem_g)
            pltpu.async_copy(att_hbm.at[idx_v.at[slot, 1]], attd_v.at[p],
                             sem_g)

        def drain_gathers(p):
            pltpu.make_async_copy(xw_hbm.at[idx_v.at[0, 0]], rows_v.at[p],
                                  sem_g).wait()
            pltpu.make_async_copy(att_hbm.at[idx_v.at[0, 0]], atts_v.at[p],
                                  sem_g).wait()
            pltpu.make_async_copy(att_hbm.at[idx_v.at[0, 1]], attd_v.at[p],
                                  sem_g).wait()

        def issue_scatters(slot, p):
            pltpu.async_copy(rows_v.at[p], acc_num.at[idx_v.at[slot, 1]],
                             sem_s, add=True)
            pltpu.async_copy(ex_v.at[p], acc_den.at[idx_v.at[slot, 1]],
                             sem_s, add=True)

        def drain_scatters(p):
            pltpu.make_async_copy(rows_v.at[p], acc_num.at[idx_v.at[0, 1]],
                                  sem_s).wait()
            pltpu.make_async_copy(ex_v.at[p], acc_den.at[idx_v.at[0, 1]],
                                  sem_s).wait()

        def compute(p):
            atts_p, attd_p = atts_v.at[p], attd_v.at[p]
            ex_p, rows_p = ex_v.at[p], rows_v.at[p]

            def _grp(g, _):
                rid = g * 16 + lanes
                for h in range(heads):
                    hs = jnp.full((16,), h, jnp.int32)
                    hd = jnp.full((16,), heads + h, jnp.int32)
                    e = (plsc.load_gather(atts_p, [rid, hs])
                         + plsc.load_gather(attd_p, [rid, hd]))
                    e = jnp.where(e < 0, e * jnp.float32(0.2), e)
                    plsc.store_scatter(ex_p, [rid, hs], jnp.exp(e))
                return 0
            lax.fori_loop(0, CH // 16, _grp, 0)

            lph = width // heads // 16

            def _edge(i, _):
                exrow = ex_p[i, pl.ds(0, 16)]
                for h in range(heads):
                    m = exrow[h]
                    for v in range(lph):
                        off = (h * lph + v) * 16
                        rows_p[i, pl.ds(off, 16)] = (
                            rows_p[i, pl.ds(off, 16)] * m)
                return 0
            lax.fori_loop(0, CH, _edge, 0)

        pltpu.sync_copy(srcs_hbm.at[w, 0], idx_v.at[0, 0])
        pltpu.sync_copy(dsts_hbm.at[w, 0], idx_v.at[0, 1])
        issue_idx(1, 1)
        issue_gathers(0, 0)

        def _iter(j, _):
            p = lax.rem(j, 2)
            drain_gathers(p)
            pl.when(j >= 1)(lambda: drain_scatters(1 - p))

            def _pref():
                drain_idx(lax.rem(j + 1, 3))
                issue_gathers(lax.rem(j + 1, 3), 1 - p)
            pl.when(j + 1 < K)(_pref)
            pl.when(j + 2 < K)(
                lambda: issue_idx(j + 2, lax.rem(j + 2, 3)))
            compute(p)
            issue_scatters(lax.rem(j, 3), p)
            return 0
        lax.fori_loop(0, K, _iter, 0)
        drain_scatters(lax.rem(K - 1, 2))

        plsc.subcore_barrier()
        pltpu.sync_copy(acc_num.at[pl.ds(s * RT, RT)],
                        num_out.at[c, pl.ds(s * RT, RT)])
        pltpu.sync_copy(acc_den.at[pl.ds(s * RT, RT)],
                        den_out.at[c, pl.ds(s * RT, RT)])

    return kfn


def _tc_a(x_ref, w1_ref, a1_ref, xw_ref, att_ref):
    xw = jnp.dot(x_ref[...], w1_ref[...], preferred_element_type=jnp.float32)
    xw_ref[...] = xw
    att_ref[...] = jnp.dot(xw, a1_ref[...], preferred_element_type=jnp.float32)


def _tc_b(n0, n1, d0, d1, b1r, w2, a2, bh, xw2_ref, att2_ref):
    num = n0[...] + n1[...]
    den = jnp.dot(d0[...] + d1[...], bh[...],
                  preferred_element_type=jnp.float32)
    h = num / (den + jnp.float32(1e-16)) + b1r[...]
    h = jnp.where(h > 0, h, jnp.exp(h) - jnp.float32(1.0))
    xw2 = jnp.dot(h, w2[...], preferred_element_type=jnp.float32)
    xw2_ref[...] = xw2
    att2_ref[...] = jnp.dot(xw2, a2[...], preferred_element_type=jnp.float32)


def _tc_c(n0, n1, d0, d1, b2r, b2m, wh_ref, bhr, out_ref):
    den = jnp.dot(d0[...] + d1[...], b2m[...],
                  preferred_element_type=jnp.float32)
    h = (n0[...] + n1[...]) / (den + jnp.float32(1e-16)) + b2r[...]
    h = jnp.where(h > 0, h, jnp.exp(h) - jnp.float32(1.0))
    out_ref[...] = jnp.dot(h, wh_ref[...],
                           preferred_element_type=jnp.float32) + bhr[...]


def _full(shape):
    return pl.BlockSpec(shape, lambda i: tuple(0 for _ in shape))


def _rows(width):
    return pl.BlockSpec((BM, width), lambda i: (i, 0))


def kernel(x, edge_index, W1, a_src1, a_dst1, b1, W2, a_src2, a_dst2, b2,
           Wh, bh):
    xp = jnp.zeros((NP, D), jnp.float32).at[:N].set(x)

    eye8 = jnp.eye(8, dtype=jnp.float32)
    a_s = (eye8[:, None, :] * a_src1[:, :, None]).reshape(128, 8)
    a_d = (eye8[:, None, :] * a_dst1[:, :, None]).reshape(128, 8)
    A1 = jnp.concatenate([a_s, a_d], axis=1)
    BH = jnp.concatenate([jnp.kron(eye8, jnp.ones((1, 16), jnp.float32)),
                          jnp.zeros((8, 128), jnp.float32)])
    A2 = (jnp.zeros((32, 16), jnp.float32)
          .at[:, 0].set(a_src2[0]).at[:, 1].set(a_dst2[0]))
    B2 = jnp.zeros((16, 32), jnp.float32).at[0, :].set(1.0)
    Wh8 = jnp.zeros((32, 8), jnp.float32).at[:, :4].set(Wh)
    bh8 = jnp.zeros((1, 8), jnp.float32).at[0, :4].set(bh)
    b1r = b1.reshape(1, 128)
    b2r = b2.reshape(1, 32)

    E = edge_index.shape[1]
    e_tot = E + N
    n_chunks = -(-e_tot // (NW * CH))
    e_pad = n_chunks * NW * CH
    loop = jnp.arange(N, dtype=jnp.int32)
    pad = (jnp.arange(e_pad - e_tot, dtype=jnp.int32) % (NP - N)) + N
    srcs = jnp.concatenate([edge_index[0], loop, pad]).reshape(NW, n_chunks, CH)
    dsts = jnp.concatenate([edge_index[1], loop, pad]).reshape(NW, n_chunks, CH)

    xw, att = pl.pallas_call(
        _tc_a,
        grid=(NP // BM,),
        in_specs=[_rows(128), _full((128, 128)), _full((128, 16))],
        out_specs=[_rows(128), _rows(16)],
        out_shape=[_f32(NP, 128), _f32(NP, 16)],
    )(xp, W1, A1)

    num, den = _edge_kernel(n_chunks, 128, 8)(xw, att, srcs, dsts)

    xw2, att2 = pl.pallas_call(
        _tc_b,
        grid=(NP // BM,),
        in_specs=[_rows(128), _rows(128), _rows(16), _rows(16),
                  _full((1, 128)), _full((128, 32)), _full((32, 16)),
                  _full((16, 128))],
        out_specs=[_rows(32), _rows(16)],
        out_shape=[_f32(NP, 32), _f32(NP, 16)],
    )(num[0], num[1], den[0], den[1], b1r, W2, A2, BH)

    num2, den2 = _edge_kernel(n_chunks, 32, 1)(xw2, att2, srcs, dsts)

    out = pl.pallas_call(
        _tc_c,
        grid=(NP // BM,),
        in_specs=[_rows(32), _rows(32), _rows(16), _rows(16),
                  _full((1, 32)), _full((16, 32)), _full((32, 8)),
                  _full((1, 8))],
        out_specs=_rows(8),
        out_shape=_f32(NP, 8),
    )(num2[0], num2[1], den2[0], den2[1], b2r, B2, Wh8, bh8)

    return out[:N, :4]


# speedup vs baseline: 1.0012x; 1.0012x over previous
import functools

import jax
import jax.numpy as jnp
from jax import lax
from jax.experimental import pallas as pl
from jax.experimental.pallas import tpu as pltpu
from jax.experimental.pallas import tpu_sc as plsc

N = 10000
D = 128
NP = 10240
NC, NS = 2, 16
NW = NC * NS
CH = 96
RT = NP // NS
ZB = 64
BM = 512


def _f32(*shape):
    return jax.ShapeDtypeStruct(shape, jnp.float32)


def _edge_kernel(n_chunks, width, heads):
    mesh = plsc.VectorSubcoreMesh(core_axis_name="c", subcore_axis_name="s")

    @functools.partial(
        pl.kernel,
        out_type=[_f32(NC, NP, width), _f32(NC, NP, 16)],
        mesh=mesh,
        compiler_params=pltpu.CompilerParams(
            needs_layout_passes=False, use_tc_tiling_on_sc=False),
        scratch_types=[
            pltpu.VMEM_SHARED((NP, width), jnp.float32),
            pltpu.VMEM_SHARED((NP, 16), jnp.float32),
            pltpu.VMEM((3, 2, CH), jnp.int32),
            pltpu.VMEM((2, CH, width), jnp.float32),
            pltpu.VMEM((2, CH, 16), jnp.float32),
            pltpu.VMEM((2, CH, 16), jnp.float32),
            pltpu.VMEM((2, CH, 16), jnp.float32),
            pltpu.SemaphoreType.DMA,
            pltpu.SemaphoreType.DMA,
            pltpu.SemaphoreType.DMA,
        ],
    )
    def kfn(xw_hbm, att_hbm, srcs_hbm, dsts_hbm, num_out, den_out,
            acc_num, acc_den, idx_v, rows_v, atts_v, attd_v, ex_v,
            sem_g, sem_s, sem_i):
        c = lax.axis_index("c")
        s = lax.axis_index("s")
        w = s * NC + c
        K = n_chunks
        zero16 = jnp.zeros((16,), jnp.float32)
        lanes = lax.iota(jnp.int32, 16)

        def _zrow(r, _):
            for j in range(width // 16):
                rows_v[0, r, pl.ds(j * 16, 16)] = zero16
            ex_v[0, r, pl.ds(0, 16)] = zero16
            ex_v[1, r, pl.ds(0, 16)] = zero16
            return 0
        lax.fori_loop(0, CH, _zrow, 0)

        for i in range(RT // ZB):
            base = s * RT + i * ZB
            pltpu.sync_copy(rows_v.at[0, pl.ds(0, ZB)],
                            acc_num.at[pl.ds(base, ZB)])
            pltpu.sync_copy(ex_v.at[0, pl.ds(0, ZB)],
                            acc_den.at[pl.ds(base, ZB)])
        plsc.subcore_barrier()

        def issue_idx(jj, slot):
            pltpu.async_copy(srcs_hbm.at[w, jj], idx_v.at[slot, 0], sem_i)
            pltpu.async_copy(dsts_hbm.at[w, jj], idx_v.at[slot, 1], sem_i)

        def drain_idx(slot):
            pltpu.make_async_copy(srcs_hbm.at[w, 0], idx_v.at[slot, 0],
                                  sem_i).wait()
            pltpu.make_async_copy(srcs_hbm.at[w, 0], idx_v.at[slot, 1],
                                  sem_i).wait()

        def issue_gathers(slot, p):
            pltpu.async_copy(xw_hbm.at[idx_v.at[slot, 0]], rows_v.at[p],
                             sem_g)
            pltpu.async_copy(att_hbm.at[idx_v.at[slot, 0]], atts_v.at[p],
                             sem_g)
            pltpu.async_copy(att_hbm.at[idx_v.at[slot, 1]], attd_v.at[p],
                             sem_g)

        def drain_gathers(p):
            pltpu.make_async_copy(xw_hbm.at[idx_v.at[0, 0]], rows_v.at[p],
                                  sem_g).wait()
            pltpu.make_async_copy(att_hbm.at[idx_v.at[0, 0]], atts_v.at[p],
                                  sem_g).wait()
            pltpu.make_async_copy(att_hbm.at[idx_v.at[0, 1]], attd_v.at[p],
                                  sem_g).wait()

        def issue_scatters(slot, p):
            pltpu.async_copy(rows_v.at[p], acc_num.at[idx_v.at[slot, 1]],
                             sem_s, add=True)
            pltpu.async_copy(ex_v.at[p], acc_den.at[idx_v.at[slot, 1]],
                             sem_s, add=True)

        def drain_scatters(p):
            pltpu.make_async_copy(rows_v.at[p], acc_num.at[idx_v.at[0, 1]],
                                  sem_s).wait()
            pltpu.make_async_copy(ex_v.at[p], acc_den.at[idx_v.at[0, 1]],
                                  sem_s).wait()

        def compute(p):
            atts_p, attd_p = atts_v.at[p], attd_v.at[p]
            ex_p, rows_p = ex_v.at[p], rows_v.at[p]

            def _grp(g, _):
                rid = g * 16 + lanes
                for h in range(heads):
                    hs = jnp.full((16,), h, jnp.int32)
                    hd = jnp.full((16,), heads + h, jnp.int32)
                    e = (plsc.load_gather(atts_p, [rid, hs])
                         + plsc.load_gather(attd_p, [rid, hd]))
                    e = jnp.where(e < 0, e * jnp.float32(0.2), e)
                    plsc.store_scatter(ex_p, [rid, hs], jnp.exp(e))
                return 0
            lax.fori_loop(0, CH // 16, _grp, 0)

            lph = width // heads // 16

            def _edge(i, _):
                exrow = ex_p[i, pl.ds(0, 16)]
                for h in range(heads):
                    m = exrow[h]
                    for v in range(lph):
                        off = (h * lph + v) * 16
                        rows_p[i, pl.ds(off, 16)] = (
                            rows_p[i, pl.ds(off, 16)] * m)
                return 0
            lax.fori_loop(0, CH, _edge, 0)

        pltpu.sync_copy(srcs_hbm.at[w, 0], idx_v.at[0, 0])
        pltpu.sync_copy(dsts_hbm.at[w, 0], idx_v.at[0, 1])
        issue_idx(1, 1)
        issue_gathers(0, 0)

        def _iter(j, _):
            p = lax.rem(j, 2)
            drain_gathers(p)
            pl.when(j >= 1)(lambda: drain_scatters(1 - p))

            def _pref():
                drain_idx(lax.rem(j + 1, 3))
                issue_gathers(lax.rem(j + 1, 3), 1 - p)
            pl.when(j + 1 < K)(_pref)
            pl.when(j + 2 < K)(
                lambda: issue_idx(j + 2, lax.rem(j + 2, 3)))
            compute(p)
            issue_scatters(lax.rem(j, 3), p)
            return 0
        lax.fori_loop(0, K, _iter, 0)
        drain_scatters(lax.rem(K - 1, 2))

        plsc.subcore_barrier()
        pltpu.sync_copy(acc_num.at[pl.ds(s * RT, RT)],
                        num_out.at[c, pl.ds(s * RT, RT)])
        pltpu.sync_copy(acc_den.at[pl.ds(s * RT, RT)],
                        den_out.at[c, pl.ds(s * RT, RT)])

    return kfn


def _tc_a(x_ref, w1_ref, a1_ref, xw_ref, att_ref):
    xw = jnp.dot(x_ref[...], w1_ref[...], preferred_element_type=jnp.float32)
    xw_ref[...] = xw
    att_ref[...] = jnp.dot(xw, a1_ref[...], preferred_element_type=jnp.float32)


def _tc_b(n0, n1, d0, d1, b1r, w2, a2, bh, xw2_ref, att2_ref):
    num = n0[...] + n1[...]
    den = jnp.dot(d0[...] + d1[...], bh[...],
                  preferred_element_type=jnp.float32)
    h = num / (den + jnp.float32(1e-16)) + b1r[...]
    h = jnp.where(h > 0, h, jnp.exp(h) - jnp.float32(1.0))
    xw2 = jnp.dot(h, w2[...], preferred_element_type=jnp.float32)
    xw2_ref[...] = xw2
    att2_ref[...] = jnp.dot(xw2, a2[...], preferred_element_type=jnp.float32)


def _tc_c(n0, n1, d0, d1, b2r, b2m, wh_ref, bhr, out_ref):
    den = jnp.dot(d0[...] + d1[...], b2m[...],
                  preferred_element_type=jnp.float32)
    h = (n0[...] + n1[...]) / (den + jnp.float32(1e-16)) + b2r[...]
    h = jnp.where(h > 0, h, jnp.exp(h) - jnp.float32(1.0))
    out_ref[...] = jnp.dot(h, wh_ref[...],
                           preferred_element_type=jnp.float32) + bhr[...]


def _full(shape):
    return pl.BlockSpec(shape, lambda i: tuple(0 for _ in shape))


def _rows(width):
    return pl.BlockSpec((BM, width), lambda i: (i, 0))


def kernel(x, edge_index, W1, a_src1, a_dst1, b1, W2, a_src2, a_dst2, b2,
           Wh, bh):
    xp = jnp.zeros((NP, D), jnp.float32).at[:N].set(x)

    eye8 = jnp.eye(8, dtype=jnp.float32)
    a_s = (eye8[:, None, :] * a_src1[:, :, None]).reshape(128, 8)
    a_d = (eye8[:, None, :] * a_dst1[:, :, None]).reshape(128, 8)
    A1 = jnp.concatenate([a_s, a_d], axis=1)
    BH = jnp.concatenate([jnp.kron(eye8, jnp.ones((1, 16), jnp.float32)),
                          jnp.zeros((8, 128), jnp.float32)])
    A2 = (jnp.zeros((32, 16), jnp.float32)
          .at[:, 0].set(a_src2[0]).at[:, 1].set(a_dst2[0]))
    B2 = jnp.zeros((16, 32), jnp.float32).at[0, :].set(1.0)
    Wh8 = jnp.zeros((32, 8), jnp.float32).at[:, :4].set(Wh)
    bh8 = jnp.zeros((1, 8), jnp.float32).at[0, :4].set(bh)
    b1r = b1.reshape(1, 128)
    b2r = b2.reshape(1, 32)

    E = edge_index.shape[1]
    e_tot = E + N
    n_chunks = -(-e_tot // (NW * CH))
    e_pad = n_chunks * NW * CH
    loop = jnp.arange(N, dtype=jnp.int32)
    pad = (jnp.arange(e_pad - e_tot, dtype=jnp.int32) % (NP - N)) + N
    srcs = jnp.concatenate([edge_index[0], loop, pad]).reshape(NW, n_chunks, CH)
    dsts = jnp.concatenate([edge_index[1], loop, pad]).reshape(NW, n_chunks, CH)

    xw, att = pl.pallas_call(
        _tc_a,
        grid=(NP // BM,),
        in_specs=[_rows(128), _full((128, 128)), _full((128, 16))],
        out_specs=[_rows(128), _rows(16)],
        out_shape=[_f32(NP, 128), _f32(NP, 16)],
    )(xp, W1, A1)

    num, den = _edge_kernel(n_chunks, 128, 8)(xw, att, srcs, dsts)

    xw2, att2 = pl.pallas_call(
        _tc_b,
        grid=(NP // BM,),
        in_specs=[_rows(128), _rows(128), _rows(16), _rows(16),
                  _full((1, 128)), _full((128, 32)), _full((32, 16)),
                  _full((16, 128))],
        out_specs=[_rows(32), _rows(16)],
        out_shape=[_f32(NP, 32), _f32(NP, 16)],
    )(num[0], num[1], den[0], den[1], b1r, W2, A2, BH)

    num2, den2 = _edge_kernel(n_chunks, 32, 1)(xw2, att2, srcs, dsts)

    out = pl.pallas_call(
        _tc_c,
        grid=(NP // BM,),
        in_specs=[_rows(32), _rows(32), _rows(16), _rows(16),
                  _full((1, 32)), _full((16, 32)), _full((32, 8)),
                  _full((1, 8))],
        out_specs=_rows(8),
        out_shape=_f32(NP, 8),
    )(num2[0], num2[1], den2[0], den2[1], b2r, B2, Wh8, bh8)

    return out[:N, :4]
